# baseline (speedup 1.0000x reference)
import functools

import jax
import jax.numpy as jnp
from jax.experimental import pallas as pl


def _body(L, wd_ref, st_ref, semO_ref, lngs_ref, lats_ref, dis_ref, spd_ref,
          azi_ref, spt_ref, Wbig_ref, tbl_ref, out_ref, traj_ref):
    TB = wd_ref.shape[0]
    lane = jax.lax.broadcasted_iota(jnp.int32, (TB, 55), 1)
    target = jnp.where(lane < 7, wd_ref[:, :], st_ref[:, :] + 7)
    oh = (lane == target).astype(jnp.float32)
    feats = jnp.concatenate([oh, semO_ref[:, :]], axis=1)
    traj = jnp.dot(feats, Wbig_ref[:, :], preferred_element_type=jnp.float32)
    traj_ref[:, :] = traj

    spt = spt_ref[:, :]
    embs = []
    for d in range(3):
        acc = jnp.zeros((TB, L), jnp.float32)
        for k in range(1, 9):
            acc = acc + jnp.where(spt == k, tbl_ref[k, d], 0.0)
        embs.append(acc)

    chans = [lngs_ref[:, :], lats_ref[:, :], dis_ref[:, :], spd_ref[:, :],
             azi_ref[:, :]]
    LC = 8
    trajblk = jnp.broadcast_to(traj[:, None, :], (TB, LC, 12))
    for j in range(0, L, LC):
        pieces = [c[:, j:j + LC].reshape(TB, LC, 1) for c in chans]
        pieces.append(trajblk)
        pieces += [e[:, j:j + LC].reshape(TB, LC, 1) for e in embs]
        out_ref[:, j:j + LC, :] = jnp.concatenate(pieces, axis=2)


def kernel(weekday, start_time, sem_O, lngs, lats, sem_pt, travel_dis, spd,
           azimuth, weekday_table, start_time_table, sem_pt_table, W_map):
    B, L = lngs.shape
    TB = 128
    Wbig = jnp.zeros((63, 12), jnp.float32)
    Wbig = Wbig.at[0:7, 0:3].set(weekday_table)
    Wbig = Wbig.at[7:55, 3:9].set(start_time_table)
    Wbig = Wbig.at[55:63, 9:12].set(W_map.T)
    wd2 = weekday.astype(jnp.int32).reshape(B, 1)
    st2 = start_time.astype(jnp.int32).reshape(B, 1)

    grid = (B // TB,)
    out, traj = pl.pallas_call(
        functools.partial(_body, L),
        grid=grid,
        in_specs=[
            pl.BlockSpec((TB, 1), lambda i: (i, 0)),
            pl.BlockSpec((TB, 1), lambda i: (i, 0)),
            pl.BlockSpec((TB, 8), lambda i: (i, 0)),
            pl.BlockSpec((TB, L), lambda i: (i, 0)),
            pl.BlockSpec((TB, L), lambda i: (i, 0)),
            pl.BlockSpec((TB, L), lambda i: (i, 0)),
            pl.BlockSpec((TB, L), lambda i: (i, 0)),
            pl.BlockSpec((TB, L), lambda i: (i, 0)),
            pl.BlockSpec((TB, L), lambda i: (i, 0)),
            pl.BlockSpec((63, 12), lambda i: (0, 0)),
            pl.BlockSpec((9, 3), lambda i: (0, 0)),
        ],
        out_specs=[
            pl.BlockSpec((TB, L, 20), lambda i: (i, 0, 0)),
            pl.BlockSpec((TB, 12), lambda i: (i, 0)),
        ],
        out_shape=[
            jax.ShapeDtypeStruct((B, L, 20), jnp.float32),
            jax.ShapeDtypeStruct((B, 12), jnp.float32),
        ],
    )(wd2, st2, sem_O, lngs, lats, travel_dis, spd, azimuth,
      sem_pt.astype(jnp.int32), Wbig, sem_pt_table)
    return out, traj


# speedup vs baseline: 2.5012x; 2.5012x over previous
import functools

import jax
import jax.numpy as jnp
from jax.experimental import pallas as pl


def _body(L, wd_ref, st_ref, semO_ref, lngs_ref, lats_ref, dis_ref, spd_ref,
          azi_ref, spt_ref, Wbig_ref, tbl_ref, out_ref, traj_ref):
    TB = wd_ref.shape[0]
    lane = jax.lax.broadcasted_iota(jnp.int32, (TB, 55), 1)
    target = jnp.where(lane < 7, wd_ref[:, :], st_ref[:, :] + 7)
    oh = (lane == target).astype(jnp.float32)
    feats = jnp.concatenate([oh, semO_ref[:, :]], axis=1)
    traj = jnp.dot(feats, Wbig_ref[:, :], preferred_element_type=jnp.float32)
    traj_ref[:, :] = traj

    spt = spt_ref[:, :]
    embs = []
    for d in range(3):
        acc = jnp.zeros((TB, L), jnp.float32)
        for k in range(1, 9):
            acc = acc + jnp.where(spt == k, tbl_ref[k, d], 0.0)
        embs.append(acc)

    chans8 = [lngs_ref[:, :], lats_ref[:, :], dis_ref[:, :], spd_ref[:, :],
              azi_ref[:, :]] + embs
    j = 0
    for LC in (64, 64, 64, 8):
        stk = jnp.stack([c[:, j:j + LC] for c in chans8], axis=1)
        t = jnp.swapaxes(stk, 1, 2)
        trajblk = jnp.broadcast_to(traj[:, None, :], (TB, LC, 12))
        out_ref[:, j:j + LC, :] = jnp.concatenate(
            [t[:, :, 0:5], trajblk, t[:, :, 5:8]], axis=2)
        j += LC


def kernel(weekday, start_time, sem_O, lngs, lats, sem_pt, travel_dis, spd,
           azimuth, weekday_table, start_time_table, sem_pt_table, W_map):
    B, L = lngs.shape
    TB = 64
    Wbig = jnp.zeros((63, 12), jnp.float32)
    Wbig = Wbig.at[0:7, 0:3].set(weekday_table)
    Wbig = Wbig.at[7:55, 3:9].set(start_time_table)
    Wbig = Wbig.at[55:63, 9:12].set(W_map.T)
    wd2 = weekday.astype(jnp.int32).reshape(B, 1)
    st2 = start_time.astype(jnp.int32).reshape(B, 1)

    grid = (B // TB,)
    out, traj = pl.pallas_call(
        functools.partial(_body, L),
        grid=grid,
        in_specs=[
            pl.BlockSpec((TB, 1), lambda i: (i, 0)),
            pl.BlockSpec((TB, 1), lambda i: (i, 0)),
            pl.BlockSpec((TB, 8), lambda i: (i, 0)),
            pl.BlockSpec((TB, L), lambda i: (i, 0)),
            pl.BlockSpec((TB, L), lambda i: (i, 0)),
            pl.BlockSpec((TB, L), lambda i: (i, 0)),
            pl.BlockSpec((TB, L), lambda i: (i, 0)),
            pl.BlockSpec((TB, L), lambda i: (i, 0)),
            pl.BlockSpec((TB, L), lambda i: (i, 0)),
            pl.BlockSpec((63, 12), lambda i: (0, 0)),
            pl.BlockSpec((9, 3), lambda i: (0, 0)),
        ],
        out_specs=[
            pl.BlockSpec((TB, L, 20), lambda i: (i, 0, 0)),
            pl.BlockSpec((TB, 12), lambda i: (i, 0)),
        ],
        out_shape=[
            jax.ShapeDtypeStruct((B, L, 20), jnp.float32),
            jax.ShapeDtypeStruct((B, 12), jnp.float32),
        ],
    )(wd2, st2, sem_O, lngs, lats, travel_dis, spd, azimuth,
      sem_pt.astype(jnp.int32), Wbig, sem_pt_table)
    return out, traj


# speedup vs baseline: 2.5464x; 1.0181x over previous
import functools

import jax
import jax.numpy as jnp
from jax.experimental import pallas as pl


def _body(L, wd_ref, st_ref, semO_ref, lngs_ref, lats_ref, dis_ref, spd_ref,
          azi_ref, spt_ref, Wbig_ref, tbl_ref, out_ref, traj_ref):
    TB = wd_ref.shape[0]
    lane = jax.lax.broadcasted_iota(jnp.int32, (TB, 55), 1)
    target = jnp.where(lane < 7, wd_ref[:, :], st_ref[:, :] + 7)
    oh = (lane == target).astype(jnp.float32)
    feats = jnp.concatenate([oh, semO_ref[:, :]], axis=1)
    traj = jnp.dot(feats, Wbig_ref[:, :], preferred_element_type=jnp.float32)
    traj_ref[:, :] = traj

    spt = spt_ref[:, :]
    embs = []
    for d in range(3):
        acc = jnp.zeros((TB, L), jnp.float32)
        for k in range(1, 9):
            acc = acc + jnp.where(spt == k, tbl_ref[k, d], 0.0)
        embs.append(acc)

    chans8 = [lngs_ref[:, :], lats_ref[:, :], dis_ref[:, :], spd_ref[:, :],
              azi_ref[:, :]] + embs
    j = 0
    for LC in (64, 64, 64, 8):
        stk = jnp.stack([c[:, j:j + LC] for c in chans8], axis=1)
        t = jnp.swapaxes(stk, 1, 2)
        trajblk = jnp.broadcast_to(traj[:, None, :], (TB, LC, 12))
        out_ref[:, j:j + LC, :] = jnp.concatenate(
            [t[:, :, 0:5], trajblk, t[:, :, 5:8]], axis=2)
        j += LC


def kernel(weekday, start_time, sem_O, lngs, lats, sem_pt, travel_dis, spd,
           azimuth, weekday_table, start_time_table, sem_pt_table, W_map):
    B, L = lngs.shape
    TB = 128
    Wbig = jnp.zeros((63, 12), jnp.float32)
    Wbig = Wbig.at[0:7, 0:3].set(weekday_table)
    Wbig = Wbig.at[7:55, 3:9].set(start_time_table)
    Wbig = Wbig.at[55:63, 9:12].set(W_map.T)
    wd2 = weekday.astype(jnp.int32).reshape(B, 1)
    st2 = start_time.astype(jnp.int32).reshape(B, 1)

    grid = (B // TB,)
    out, traj = pl.pallas_call(
        functools.partial(_body, L),
        grid=grid,
        in_specs=[
            pl.BlockSpec((TB, 1), lambda i: (i, 0)),
            pl.BlockSpec((TB, 1), lambda i: (i, 0)),
            pl.BlockSpec((TB, 8), lambda i: (i, 0)),
            pl.BlockSpec((TB, L), lambda i: (i, 0)),
            pl.BlockSpec((TB, L), lambda i: (i, 0)),
            pl.BlockSpec((TB, L), lambda i: (i, 0)),
            pl.BlockSpec((TB, L), lambda i: (i, 0)),
            pl.BlockSpec((TB, L), lambda i: (i, 0)),
            pl.BlockSpec((TB, L), lambda i: (i, 0)),
            pl.BlockSpec((63, 12), lambda i: (0, 0)),
            pl.BlockSpec((9, 3), lambda i: (0, 0)),
        ],
        out_specs=[
            pl.BlockSpec((TB, L, 20), lambda i: (i, 0, 0)),
            pl.BlockSpec((TB, 12), lambda i: (i, 0)),
        ],
        out_shape=[
            jax.ShapeDtypeStruct((B, L, 20), jnp.float32),
            jax.ShapeDtypeStruct((B, 12), jnp.float32),
        ],
    )(wd2, st2, sem_O, lngs, lats, travel_dis, spd, azimuth,
      sem_pt.astype(jnp.int32), Wbig, sem_pt_table)
    return out, traj


# speedup vs baseline: 2.5824x; 1.0141x over previous
import functools

import jax
import jax.numpy as jnp
from jax.experimental import pallas as pl


def _body(L, wd_ref, st_ref, semO_ref, lngs_ref, lats_ref, dis_ref, spd_ref,
          azi_ref, spt_ref, Wbig_ref, tbl_ref, out_ref, traj_ref):
    TB = wd_ref.shape[0]
    lane = jax.lax.broadcasted_iota(jnp.int32, (TB, 55), 1)
    target = jnp.where(lane < 7, wd_ref[:, :], st_ref[:, :] + 7)
    oh = (lane == target).astype(jnp.float32)
    feats = jnp.concatenate([oh, semO_ref[:, :]], axis=1)
    traj = jnp.dot(feats, Wbig_ref[:, :], preferred_element_type=jnp.float32)
    traj_ref[:, :] = traj

    spt = spt_ref[:, :]
    embs = []
    for d in range(3):
        acc = jnp.zeros((TB, L), jnp.float32)
        for k in range(1, 9):
            acc = acc + jnp.where(spt == k, tbl_ref[k, d], 0.0)
        embs.append(acc)

    chans8 = [lngs_ref[:, :], lats_ref[:, :], dis_ref[:, :], spd_ref[:, :],
              azi_ref[:, :]] + embs
    j = 0
    for LC in (128, 64, 8):
        stk = jnp.stack([c[:, j:j + LC] for c in chans8], axis=1)
        t = jnp.swapaxes(stk, 1, 2)
        trajblk = jnp.broadcast_to(traj[:, None, :], (TB, LC, 12))
        out_ref[:, j:j + LC, :] = jnp.concatenate(
            [t[:, :, 0:5], trajblk, t[:, :, 5:8]], axis=2)
        j += LC


def kernel(weekday, start_time, sem_O, lngs, lats, sem_pt, travel_dis, spd,
           azimuth, weekday_table, start_time_table, sem_pt_table, W_map):
    B, L = lngs.shape
    TB = 128
    Wbig = jnp.zeros((63, 12), jnp.float32)
    Wbig = Wbig.at[0:7, 0:3].set(weekday_table)
    Wbig = Wbig.at[7:55, 3:9].set(start_time_table)
    Wbig = Wbig.at[55:63, 9:12].set(W_map.T)
    wd2 = weekday.astype(jnp.int32).reshape(B, 1)
    st2 = start_time.astype(jnp.int32).reshape(B, 1)

    grid = (B // TB,)
    out, traj = pl.pallas_call(
        functools.partial(_body, L),
        grid=grid,
        in_specs=[
            pl.BlockSpec((TB, 1), lambda i: (i, 0)),
            pl.BlockSpec((TB, 1), lambda i: (i, 0)),
            pl.BlockSpec((TB, 8), lambda i: (i, 0)),
            pl.BlockSpec((TB, L), lambda i: (i, 0)),
            pl.BlockSpec((TB, L), lambda i: (i, 0)),
            pl.BlockSpec((TB, L), lambda i: (i, 0)),
            pl.BlockSpec((TB, L), lambda i: (i, 0)),
            pl.BlockSpec((TB, L), lambda i: (i, 0)),
            pl.BlockSpec((TB, L), lambda i: (i, 0)),
            pl.BlockSpec((63, 12), lambda i: (0, 0)),
            pl.BlockSpec((9, 3), lambda i: (0, 0)),
        ],
        out_specs=[
            pl.BlockSpec((TB, L, 20), lambda i: (i, 0, 0)),
            pl.BlockSpec((TB, 12), lambda i: (i, 0)),
        ],
        out_shape=[
            jax.ShapeDtypeStruct((B, L, 20), jnp.float32),
            jax.ShapeDtypeStruct((B, 12), jnp.float32),
        ],
    )(wd2, st2, sem_O, lngs, lats, travel_dis, spd, azimuth,
      sem_pt.astype(jnp.int32), Wbig, sem_pt_table)
    return out, traj
